# Initial kernel scaffold; baseline (speedup 1.0000x reference)
import jax
import jax.numpy as jnp
from jax.experimental import pallas as pl


def kernel(x, dendrite_indices, synaptic_weights, synaptic_bias, cable_weights, soma_bias):
    raise NotImplementedError("write your pallas kernel here")


# speedup vs baseline: 2.8783x; 2.8783x over previous
import jax
import jax.numpy as jnp
from jax import lax
from jax.experimental import pallas as pl
from jax.experimental.pallas import tpu as pltpu

B = 2048
IN = 4096
SOMA = 1024
BR = 8
SAMP = 8
ND = SOMA * BR
SLOPE = 0.1

DBLK = 512
NB = ND // DBLK
SOMA_BLK = DBLK // BR


def _leaky(v):
    return jnp.where(v >= 0, v, SLOPE * v)


def _body(idx_ref, w_ref, bias_ref, cable_ref, sbias_ref, x_ref,
          out_d_ref, out_s_ref):
    row_iota = lax.broadcasted_iota(jnp.int32, (IN, DBLK), 0)
    s_acc = jnp.zeros((IN, DBLK), jnp.float32)
    for s in range(SAMP):
        idx_row = idx_ref[0, s, :][None, :]
        w_row = w_ref[0, s, :][None, :]
        hit = row_iota == idx_row
        s_acc = s_acc + jnp.where(hit, w_row, 0.0)

    pre = jnp.dot(x_ref[...], s_acc.astype(jnp.bfloat16),
                  preferred_element_type=jnp.float32)
    pre = pre + bias_ref[0]
    dact = _leaky(pre)
    out_d_ref[...] = dact

    d_iota = lax.broadcasted_iota(jnp.int32, (DBLK, SOMA_BLK), 0) >> 3
    m_iota = lax.broadcasted_iota(jnp.int32, (DBLK, SOMA_BLK), 1)
    c_mat = jnp.where(d_iota == m_iota, cable_ref[0], 0.0).astype(jnp.bfloat16)
    spre = jnp.dot(dact.astype(jnp.bfloat16), c_mat,
                   preferred_element_type=jnp.float32)
    spre = spre + sbias_ref[0]
    out_s_ref[0] = _leaky(spre)


def kernel(x, dendrite_indices, synaptic_weights, synaptic_bias,
           cable_weights, soma_bias):
    x_bf = x.astype(jnp.bfloat16)
    idx_t = dendrite_indices.astype(jnp.int32).T.reshape(SAMP, NB, DBLK)
    idx_t = jnp.transpose(idx_t, (1, 0, 2))
    w_t = synaptic_weights.T.reshape(SAMP, NB, DBLK)
    w_t = jnp.transpose(w_t, (1, 0, 2))
    bias_r = synaptic_bias.reshape(NB, 1, DBLK)
    cable_b = jnp.broadcast_to(
        cable_weights.reshape(NB, DBLK, 1), (NB, DBLK, SOMA_BLK))
    sbias_r = soma_bias.reshape(NB, 1, SOMA_BLK)

    grid = (NB,)
    dact_flat, soma_act = pl.pallas_call(
        _body,
        grid=grid,
        in_specs=[
            pl.BlockSpec((1, SAMP, DBLK), lambda k: (k, 0, 0)),
            pl.BlockSpec((1, SAMP, DBLK), lambda k: (k, 0, 0)),
            pl.BlockSpec((1, 1, DBLK), lambda k: (k, 0, 0)),
            pl.BlockSpec((1, DBLK, SOMA_BLK), lambda k: (k, 0, 0)),
            pl.BlockSpec((1, 1, SOMA_BLK), lambda k: (k, 0, 0)),
            pl.BlockSpec((B, IN), lambda k: (0, 0)),
        ],
        out_specs=[
            pl.BlockSpec((B, DBLK), lambda k: (0, k)),
            pl.BlockSpec((1, B, SOMA_BLK), lambda k: (k, 0, 0)),
        ],
        out_shape=[
            jax.ShapeDtypeStruct((B, ND), jnp.float32),
            jax.ShapeDtypeStruct((NB, B, SOMA_BLK), jnp.float32),
        ],
    )(idx_t, w_t, bias_r, cable_b, sbias_r, x_bf)

    soma_out = jnp.transpose(soma_act, (1, 0, 2)).reshape(B, SOMA)
    return soma_out, dact_flat.reshape(B, SOMA, BR)
